# baseline (speedup 1.0000x reference)
import functools

import jax
import jax.numpy as jnp
from jax import lax
from jax.experimental import pallas as pl
from jax.experimental.pallas import tpu as pltpu
from jax.experimental.pallas import tpu_sc as plsc

B, C_IN, C_OUT, N, K, D_POS = 1, 128, 128, 10000, 32, 3
NC, NS, L = 2, 16, 16
N_PAD = 10240
P_SC = N_PAD // NC
CPT = C_IN // NS
ACT = CPT * (1 + D_POS)
PBLK = 128
NCHUNK = P_SC // PBLK
BN = 512


def _sc_body(f_hbm, p_hbm, nbr_hbm, a_hbm, f_v, p_v, nbr_v, abuf_v,
             semn0, semn1, semf0, semf1):
    sc = lax.axis_index("c")
    tid = lax.axis_index("s")
    pltpu.sync_copy(f_hbm.at[pl.ds(tid * CPT, CPT)], f_v)
    pltpu.sync_copy(p_hbm, p_v)
    pt_base = sc * P_SC
    semn = (semn0, semn1)
    semf = (semf0, semf1)

    cidx = [jnp.full((L,), c, jnp.int32) for c in range(CPT)]
    didx = [jnp.full((L,), d, jnp.int32) for d in range(D_POS)]

    def _nbr_start(ch, buf):
        pltpu.async_copy(nbr_hbm.at[:, pl.ds(pt_base + ch * PBLK, PBLK)],
                         nbr_v.at[buf], semn[buf])

    def _nbr_wait(ch, buf):
        pltpu.make_async_copy(nbr_hbm.at[:, pl.ds(pt_base + ch * PBLK, PBLK)],
                              nbr_v.at[buf], semn[buf]).wait()

    def _flush_descs(ch, buf):
        pt0 = pt_base + ch * PBLK
        d0 = pltpu.make_async_copy(
            abuf_v.at[buf, pl.ds(0, CPT)],
            a_hbm.at[pl.ds(tid * CPT, CPT), pl.ds(pt0, PBLK)], semf[buf])
        d1 = pltpu.make_async_copy(
            abuf_v.at[buf, pl.ds(CPT, D_POS * CPT)],
            a_hbm.at[pl.ds(C_IN + tid * D_POS * CPT, D_POS * CPT),
                     pl.ds(pt0, PBLK)], semf[buf])
        return d0, d1

    def _block16(nbuf, b16):
        zero = jnp.zeros((L,), jnp.float32)
        s_acc = [zero] * CPT
        u_acc = [[zero] * CPT for _ in range(D_POS)]
        for k in range(K):
            jk = nbr_v[nbuf, k, pl.ds(b16 * L, L)]
            p = [plsc.load_gather(p_v, [didx[d], jk]) for d in range(D_POS)]
            for c in range(CPT):
                f = plsc.load_gather(f_v, [cidx[c], jk])
                s_acc[c] = s_acc[c] + f
                for d in range(D_POS):
                    u_acc[d][c] = u_acc[d][c] + p[d] * f
        for c in range(CPT):
            abuf_v[nbuf, c, pl.ds(b16 * L, L)] = s_acc[c]
            for d in range(D_POS):
                abuf_v[nbuf, CPT + d * CPT + c, pl.ds(b16 * L, L)] = u_acc[d][c]

    def _compute_chunk(ch, buf):
        _nbr_wait(ch, buf)

        def _blk(b16, _):
            _block16(buf, b16)
            return 0
        lax.fori_loop(0, PBLK // L, _blk, 0)
        _nbr_start(ch + 2, buf)
        d0, d1 = _flush_descs(ch, buf)
        d0.start()
        d1.start()

    _nbr_start(0, 0)
    _nbr_start(1, 1)

    def _pair(j, _):
        ch = 2 * j

        @pl.when(j > 0)
        def _():
            da, db = _flush_descs(2 * j - 2, 0)
            da.wait()
            db.wait()
        _compute_chunk(ch, 0)

        @pl.when(j > 0)
        def _():
            da, db = _flush_descs(2 * j - 1, 1)
            da.wait()
            db.wait()
        _compute_chunk(ch + 1, 1)
        return 0

    lax.fori_loop(0, NCHUNK // 2, _pair, 0)
    da, db = _flush_descs(NCHUNK - 2, 0)
    da.wait()
    db.wait()
    da, db = _flush_descs(NCHUNK - 1, 1)
    da.wait()
    db.wait()
    _nbr_wait(NCHUNK, 0)
    _nbr_wait(NCHUNK + 1, 1)


def _sc_segment_sums(f, p, nbr_pad):
    mesh = plsc.VectorSubcoreMesh(core_axis_name="c", subcore_axis_name="s",
                                  num_cores=NC, num_subcores=NS)
    return pl.kernel(
        _sc_body,
        out_type=jax.ShapeDtypeStruct((C_IN * (1 + D_POS), N_PAD), jnp.float32),
        mesh=mesh,
        compiler_params=pltpu.CompilerParams(use_tc_tiling_on_sc=False,
                                             needs_layout_passes=False),
        scratch_types=[
            pltpu.VMEM((CPT, N), jnp.float32),
            pltpu.VMEM((D_POS, N), jnp.float32),
            pltpu.VMEM((2, K, PBLK), jnp.int32),
            pltpu.VMEM((2, ACT, PBLK), jnp.float32),
            pltpu.SemaphoreType.DMA,
            pltpu.SemaphoreType.DMA,
            pltpu.SemaphoreType.DMA,
            pltpu.SemaphoreType.DMA,
        ],
    )(f, p, nbr_pad)


def _tc_body(a_ref, w_ref, th_ref, p_ref, b_ref, o_ref):
    a = a_ref[...]
    y = jnp.dot(w_ref[...], a, preferred_element_type=jnp.float32)
    z = jnp.dot(th_ref[...], a[:C_IN, :], preferred_element_type=jnp.float32)
    for d in range(D_POS):
        y = y - p_ref[d:d + 1, :] * z[d * C_OUT:(d + 1) * C_OUT, :]
    o_ref[...] = y + b_ref[:, 0:1]


def _tc_combine(a, w_all_t, th_t, p_pad, bias_col):
    return pl.pallas_call(
        _tc_body,
        grid=(N_PAD // BN,),
        in_specs=[
            pl.BlockSpec((C_IN * (1 + D_POS), BN), lambda i: (0, i)),
            pl.BlockSpec((C_OUT, C_IN * (1 + D_POS)), lambda i: (0, 0)),
            pl.BlockSpec((D_POS * C_OUT, C_IN), lambda i: (0, 0)),
            pl.BlockSpec((8, BN), lambda i: (0, i)),
            pl.BlockSpec((C_OUT, 8), lambda i: (0, 0)),
        ],
        out_specs=pl.BlockSpec((C_OUT, BN), lambda i: (0, i)),
        out_shape=jax.ShapeDtypeStruct((C_OUT, N_PAD), jnp.float32),
    )(a, w_all_t, th_t, p_pad, bias_col)


def kernel(features, weight_theta, weight_bias, bias, neighborhood, positions):
    f = features[0]
    p = positions[0]
    nbr_pad = jnp.zeros((K, N_PAD + 2 * PBLK), jnp.int32).at[:, :N].set(neighborhood[0])

    a = _sc_segment_sums(f, p, nbr_pad)

    th_u = jnp.transpose(weight_theta, (1, 0, 2)).reshape(NS, CPT, D_POS, C_OUT)
    th_u = jnp.transpose(th_u, (0, 2, 1, 3)).reshape(D_POS * C_IN, C_OUT)
    w_all_t = jnp.concatenate([weight_bias, th_u], axis=0).T
    th_t = jnp.transpose(weight_theta, (0, 2, 1)).reshape(D_POS * C_OUT, C_IN)
    p_pad = jnp.zeros((8, N_PAD), jnp.float32).at[:D_POS, :N].set(p)
    bias_col = jnp.broadcast_to(bias[:, None], (C_OUT, 8))

    out_t = _tc_combine(a, w_all_t, th_t, p_pad, bias_col)
    return out_t[:, :N][None]


# speedup vs baseline: 1.0021x; 1.0021x over previous
import functools

import jax
import jax.numpy as jnp
from jax import lax
from jax.experimental import pallas as pl
from jax.experimental.pallas import tpu as pltpu
from jax.experimental.pallas import tpu_sc as plsc

B, C_IN, C_OUT, N, K, D_POS = 1, 128, 128, 10000, 32, 3
NC, NS, L = 2, 16, 16
N_PAD = 10240
P_SC = N_PAD // NC
CPT = C_IN // NS
ACT = CPT * (1 + D_POS)
PBLK = 128
NCHUNK = P_SC // PBLK
BN = 512


def _sc_body(f_hbm, p_hbm, nbr_hbm, a_hbm, f_v, p_v, nbr_v, abuf_v,
             semn0, semn1, semf0, semf1):
    sc = lax.axis_index("c")
    tid = lax.axis_index("s")
    pltpu.sync_copy(f_hbm.at[pl.ds(tid * CPT, CPT)], f_v)
    pltpu.sync_copy(p_hbm, p_v)
    pt_base = sc * P_SC
    semn = (semn0, semn1)
    semf = (semf0, semf1)

    cidx = [jnp.full((L,), c, jnp.int32) for c in range(CPT)]
    didx = [jnp.full((L,), d, jnp.int32) for d in range(D_POS)]

    def _nbr_start(ch, buf):
        pltpu.async_copy(nbr_hbm.at[:, pl.ds(pt_base + ch * PBLK, PBLK)],
                         nbr_v.at[buf], semn[buf])

    def _nbr_wait(ch, buf):
        pltpu.make_async_copy(nbr_hbm.at[:, pl.ds(pt_base + ch * PBLK, PBLK)],
                              nbr_v.at[buf], semn[buf]).wait()

    def _flush_descs(ch, buf):
        pt0 = pt_base + ch * PBLK
        d0 = pltpu.make_async_copy(
            abuf_v.at[buf, pl.ds(0, CPT)],
            a_hbm.at[pl.ds(tid * CPT, CPT), pl.ds(pt0, PBLK)], semf[buf])
        d1 = pltpu.make_async_copy(
            abuf_v.at[buf, pl.ds(CPT, D_POS * CPT)],
            a_hbm.at[pl.ds(C_IN + tid * D_POS * CPT, D_POS * CPT),
                     pl.ds(pt0, PBLK)], semf[buf])
        return d0, d1

    def _block16(nbuf, b16):
        zero = jnp.zeros((L,), jnp.float32)
        s_acc = [zero] * CPT
        u_acc = [[zero] * CPT for _ in range(D_POS)]
        for k in range(K):
            jk = nbr_v[nbuf, k, pl.ds(b16 * L, L)]
            p = [plsc.load_gather(p_v, [didx[d], jk]) for d in range(D_POS)]
            for c in range(CPT):
                f = plsc.load_gather(f_v, [cidx[c], jk])
                s_acc[c] = s_acc[c] + f
                for d in range(D_POS):
                    u_acc[d][c] = u_acc[d][c] + p[d] * f
        for c in range(CPT):
            abuf_v[nbuf, c, pl.ds(b16 * L, L)] = s_acc[c]
            for d in range(D_POS):
                abuf_v[nbuf, CPT + d * CPT + c, pl.ds(b16 * L, L)] = u_acc[d][c]

    def _compute_chunk(ch, buf):
        _nbr_wait(ch, buf)

        def _blk(b16, _):
            _block16(buf, b16)
            return 0
        lax.fori_loop(0, PBLK // L, _blk, 0)
        _nbr_start(ch + 2, buf)
        d0, d1 = _flush_descs(ch, buf)
        d0.start()
        d1.start()

    _nbr_start(0, 0)
    _nbr_start(1, 1)

    def _pair(j, _):
        ch = 2 * j

        @pl.when(j > 0)
        def _():
            da, db = _flush_descs(2 * j - 2, 0)
            da.wait()
            db.wait()
        _compute_chunk(ch, 0)

        @pl.when(j > 0)
        def _():
            da, db = _flush_descs(2 * j - 1, 1)
            da.wait()
            db.wait()
        _compute_chunk(ch + 1, 1)
        return 0

    lax.fori_loop(0, NCHUNK // 2, _pair, 0)
    da, db = _flush_descs(NCHUNK - 2, 0)
    da.wait()
    db.wait()
    da, db = _flush_descs(NCHUNK - 1, 1)
    da.wait()
    db.wait()
    _nbr_wait(NCHUNK, 0)
    _nbr_wait(NCHUNK + 1, 1)


def _sc_segment_sums(f, p, nbr_pad):
    mesh = plsc.VectorSubcoreMesh(core_axis_name="c", subcore_axis_name="s",
                                  num_cores=NC, num_subcores=NS)
    return pl.kernel(
        _sc_body,
        out_type=jax.ShapeDtypeStruct((C_IN * (1 + D_POS), N_PAD), jnp.float32),
        mesh=mesh,
        compiler_params=pltpu.CompilerParams(use_tc_tiling_on_sc=False,
                                             needs_layout_passes=False),
        scratch_types=[
            pltpu.VMEM((CPT, N), jnp.float32),
            pltpu.VMEM((D_POS, N), jnp.float32),
            pltpu.VMEM((2, K, PBLK), jnp.int32),
            pltpu.VMEM((2, ACT, PBLK), jnp.float32),
            pltpu.SemaphoreType.DMA,
            pltpu.SemaphoreType.DMA,
            pltpu.SemaphoreType.DMA,
            pltpu.SemaphoreType.DMA,
        ],
    )(f, p, nbr_pad)


def _tc_body(a_ref, w_ref, th_ref, p_ref, b_ref, o_ref):
    a = a_ref[...].astype(jnp.bfloat16)
    y = jnp.dot(w_ref[...], a, preferred_element_type=jnp.float32)
    z = jnp.dot(th_ref[...], a[:C_IN, :], preferred_element_type=jnp.float32)
    for d in range(D_POS):
        y = y - p_ref[d:d + 1, :] * z[d * C_OUT:(d + 1) * C_OUT, :]
    o_ref[...] = y + b_ref[:, 0:1]


def _tc_combine(a, w_all_t, th_t, p_pad, bias_col):
    return pl.pallas_call(
        _tc_body,
        grid=(N_PAD // BN,),
        in_specs=[
            pl.BlockSpec((C_IN * (1 + D_POS), BN), lambda i: (0, i)),
            pl.BlockSpec((C_OUT, C_IN * (1 + D_POS)), lambda i: (0, 0)),
            pl.BlockSpec((D_POS * C_OUT, C_IN), lambda i: (0, 0)),
            pl.BlockSpec((8, BN), lambda i: (0, i)),
            pl.BlockSpec((C_OUT, 8), lambda i: (0, 0)),
        ],
        out_specs=pl.BlockSpec((C_OUT, BN), lambda i: (0, i)),
        out_shape=jax.ShapeDtypeStruct((C_OUT, N_PAD), jnp.float32),
    )(a, w_all_t, th_t, p_pad, bias_col)


def kernel(features, weight_theta, weight_bias, bias, neighborhood, positions):
    f = features[0]
    p = positions[0]
    nbr_pad = jnp.zeros((K, N_PAD + 2 * PBLK), jnp.int32).at[:, :N].set(neighborhood[0])

    a = _sc_segment_sums(f, p, nbr_pad)

    th_u = jnp.transpose(weight_theta, (1, 0, 2)).reshape(NS, CPT, D_POS, C_OUT)
    th_u = jnp.transpose(th_u, (0, 2, 1, 3)).reshape(D_POS * C_IN, C_OUT)
    w_all_t = jnp.concatenate([weight_bias, th_u], axis=0).T.astype(jnp.bfloat16)
    th_t = jnp.transpose(weight_theta, (0, 2, 1)).reshape(
        D_POS * C_OUT, C_IN).astype(jnp.bfloat16)
    p_pad = jnp.zeros((8, N_PAD), jnp.float32).at[:D_POS, :N].set(p)
    bias_col = jnp.broadcast_to(bias[:, None], (C_OUT, 8))

    out_t = _tc_combine(a, w_all_t, th_t, p_pad, bias_col)
    return out_t[:, :N][None]
